# baseline (speedup 1.0000x reference)
import functools

import jax
import jax.numpy as jnp
from jax import lax
from jax.experimental import pallas as pl
from jax.experimental.pallas import tpu as pltpu, tpu_sc as plsc

B_TOTAL = 4 * 8192
D_EMB = 64
NC, NS = 2, 16
NW = NC * NS
CHUNK = 128
B_PER_W = B_TOTAL // NW
N_CHUNKS = B_PER_W // CHUNK

_mesh = plsc.VectorSubcoreMesh(core_axis_name="c", subcore_axis_name="s")


@functools.partial(
    pl.kernel,
    mesh=_mesh,
    out_type=jax.ShapeDtypeStruct((B_TOTAL, D_EMB), jnp.float32),
    scratch_types=[
        pltpu.VMEM((N_CHUNKS, CHUNK), jnp.int32),
        pltpu.VMEM((B_PER_W, D_EMB), jnp.float32),
        [pltpu.SemaphoreType.DMA] * N_CHUNKS,
        pltpu.SemaphoreType.DMA,
    ],
    compiler_params=pltpu.CompilerParams(use_tc_tiling_on_sc=False),
)
def _gather(idx_hbm, table_hbm, out_hbm, idx_v, rows_v, gsems, ssem):
    wid = lax.axis_index("s") * NC + lax.axis_index("c")
    base = wid * B_PER_W
    pltpu.sync_copy(idx_hbm.at[wid], idx_v)
    gh = []
    for j in range(N_CHUNKS):
        gh.append(
            pltpu.async_copy(
                table_hbm.at[idx_v.at[j]],
                rows_v.at[pl.ds(j * CHUNK, CHUNK)],
                gsems[j],
            )
        )
    sh = []
    for j in range(N_CHUNKS):
        gh[j].wait()
        sh.append(
            pltpu.async_copy(
                rows_v.at[pl.ds(j * CHUNK, CHUNK)],
                out_hbm.at[pl.ds(base + j * CHUNK, CHUNK)],
                ssem,
            )
        )
    for h in sh:
        h.wait()


def kernel(x, table):
    idx = x.reshape(NW, N_CHUNKS, CHUNK)
    out = _gather(idx, table)
    return out.reshape(4, 8192, D_EMB)


# speedup vs baseline: 1.0055x; 1.0055x over previous
import functools

import jax
import jax.numpy as jnp
from jax import lax
from jax.experimental import pallas as pl
from jax.experimental.pallas import tpu as pltpu, tpu_sc as plsc

B_TOTAL = 4 * 8192
D_EMB = 64
NC, NS = 2, 16
NW = NC * NS
CHUNK = 128
B_PER_W = B_TOTAL // NW
N_CHUNKS = B_PER_W // CHUNK

_mesh = plsc.VectorSubcoreMesh(core_axis_name="c", subcore_axis_name="s")


@functools.partial(
    pl.kernel,
    mesh=_mesh,
    out_type=jax.ShapeDtypeStruct((B_TOTAL, D_EMB), jnp.float32),
    scratch_types=[
        pltpu.VMEM((N_CHUNKS, CHUNK), jnp.int32),
        pltpu.VMEM((B_PER_W, D_EMB), jnp.float32),
        pltpu.VMEM_SHARED((8192, D_EMB), jnp.float32),
        [pltpu.SemaphoreType.DMA] * N_CHUNKS,
        pltpu.SemaphoreType.DMA,
    ],
    compiler_params=pltpu.CompilerParams(use_tc_tiling_on_sc=False),
)
def _gather(idx_hbm, table_hbm, out_hbm, idx_v, rows_v, table_sp, gsems, ssem):
    s = lax.axis_index("s")
    wid = s * NC + lax.axis_index("c")
    base = wid * B_PER_W
    rows_per_tile = 8192 // NS
    pltpu.sync_copy(
        table_hbm.at[pl.ds(s * rows_per_tile, rows_per_tile)],
        table_sp.at[pl.ds(s * rows_per_tile, rows_per_tile)],
    )
    idxh = pltpu.async_copy(idx_hbm.at[wid], idx_v, ssem)
    plsc.subcore_barrier()
    idxh.wait()
    gh = []
    for j in range(N_CHUNKS):
        gh.append(
            pltpu.async_copy(
                table_sp.at[idx_v.at[j]],
                rows_v.at[pl.ds(j * CHUNK, CHUNK)],
                gsems[j],
            )
        )
    sh = []
    for j in range(N_CHUNKS):
        gh[j].wait()
        sh.append(
            pltpu.async_copy(
                rows_v.at[pl.ds(j * CHUNK, CHUNK)],
                out_hbm.at[pl.ds(base + j * CHUNK, CHUNK)],
                ssem,
            )
        )
    for h in sh:
        h.wait()


def kernel(x, table):
    idx = x.reshape(NW, N_CHUNKS, CHUNK)
    out = _gather(idx, table)
    return out.reshape(4, 8192, D_EMB)
